# baseline (speedup 1.0000x reference)
import functools

import jax
import jax.numpy as jnp
from jax.experimental import pallas as pl
from jax.experimental.pallas import tpu as pltpu


def _fused_kernel(xp_ref, xc_ref, xn_ref, w_ref, g_ref, b_ref,
                  wa_ref, ba_ref, wb_ref, bb_ref, wc_ref, bc_ref,
                  wcls_ref, bcls_ref,
                  fs_ref, yp_ref, yh_ref,
                  y_sc, st_sc, dc_sc, *, S, blk, nblk):
    p = pl.program_id(0)
    i = pl.program_id(1)

    @pl.when(p == 0)
    def _conv_phase():
        fea = xc_ref.shape[1]
        z8 = jnp.zeros((8, fea), jnp.float32)
        xp = xp_ref[...] * (i > 0).astype(jnp.float32)
        xn = xn_ref[...] * (i < nblk - 1).astype(jnp.float32)
        x_cat = jnp.concatenate([z8, xp, xc_ref[...], xn, z8],
                                axis=0).astype(jnp.bfloat16)
        qs = []
        for dc in range(3):
            q = None
            for dr in range(3):
                xs = x_cat[128 * dr:128 * dr + blk + 16, :]
                d = jnp.dot(xs, w_ref[dr * 3 + dc],
                            preferred_element_type=jnp.float32)
                q = d if q is None else q + d
            qs.append(q)
        c = jax.lax.broadcasted_iota(jnp.int32, (blk, 1), 0) % S
        y = (qs[1][8:8 + blk, :]
             + jnp.where(c != 0, qs[0][7:7 + blk, :], 0.0)
             + jnp.where(c != S - 1, qs[2][9:9 + blk, :], 0.0))
        y_sc[pl.ds(i * blk, blk), :] = y
        part = jnp.concatenate([jnp.sum(y, axis=0, keepdims=True),
                                jnp.sum(y * y, axis=0, keepdims=True)], axis=0)

        @pl.when(i == 0)
        def _init():
            st_sc[...] = part

        @pl.when(i > 0)
        def _acc():
            st_sc[...] = st_sc[...] + part

    @pl.when(p == 1)
    def _head_phase():
        n = nblk * blk
        mean = st_sc[0:1, :] / n
        var = st_sc[1:2, :] / n - mean * mean
        scale = jax.lax.rsqrt(var + 1e-5) * g_ref[...]
        y = y_sc[pl.ds(i * blk, blk), :]
        hh = jnp.maximum((y - mean) * scale + b_ref[...], 0.0)
        a = jnp.tanh(
            jnp.dot(hh, wa_ref[...], preferred_element_type=jnp.float32)
            + ba_ref[...])
        gate = jax.nn.sigmoid(
            jnp.dot(hh, wb_ref[...], preferred_element_type=jnp.float32)
            + bb_ref[...])
        ab = a * gate
        det = jax.lax.dot_general(
            wc_ref[...], ab, (((1,), (1,)), ((), ())),
            preferred_element_type=jnp.float32) + bc_ref[...]
        cls = jax.lax.dot_general(
            wcls_ref[...], hh, (((1,), (1,)), ((), ())),
            preferred_element_type=jnp.float32) + bcls_ref[...]
        cm = jnp.max(cls, axis=0, keepdims=True)
        ce = jnp.exp(cls - cm)
        cls_score = ce / jnp.sum(ce, axis=0, keepdims=True)
        dc_sc[0:2, pl.ds(i * blk, blk)] = det
        dc_sc[2:4, pl.ds(i * blk, blk)] = cls_score

        @pl.when(i == nblk - 1)
        def _finale():
            det_f = dc_sc[0:2, :]
            cls_f = dc_sc[2:4, :]
            dm = jnp.max(det_f, axis=1, keepdims=True)
            de = jnp.exp(det_f - dm)
            det_s = de / jnp.sum(de, axis=1, keepdims=True)
            fs = cls_f * det_s
            fs_ref[...] = fs.T
            yp = jnp.transpose(
                jnp.clip(jnp.sum(fs, axis=1, keepdims=True),
                         1e-10, 1.0 - 1e-10))
            yp_ref[...] = yp
            yh_ref[...] = (yp[:, 1:2] > yp[:, 0:1]).astype(jnp.int32)


def kernel(h, conv_w, bn_gamma, bn_beta, Wa, ba, Wb, bb, Wc, bc, Wcls, bcls):
    N, FEA = h.shape
    HID = conv_w.shape[0]
    D = Wa.shape[0]
    NC = Wc.shape[0]
    S = 128
    rows = 16
    blk = rows * S
    nblk = N // blk
    halo = 128
    nhalo = N // halo
    hpb = blk // halo
    w_taps = jnp.transpose(conv_w, (2, 3, 1, 0)).reshape(9, FEA, HID)
    w_taps = w_taps.astype(jnp.bfloat16)

    def xp_map(p, i):
        return (jnp.where(p == 0, jnp.maximum(hpb * i - 1, 0),
                          hpb * (nblk - 1) - 1), 0)

    def xc_map(p, i):
        return (jnp.where(p == 0, i, nblk - 1), 0)

    def xn_map(p, i):
        return (jnp.where(p == 0, jnp.minimum(hpb * (i + 1), nhalo - 1),
                          nhalo - 1), 0)

    const2 = lambda p, i: (0, 0)
    const3 = lambda p, i: (0, 0, 0)

    fs, yp, yh = pl.pallas_call(
        functools.partial(_fused_kernel, S=S, blk=blk, nblk=nblk),
        grid=(2, nblk),
        in_specs=[
            pl.BlockSpec((halo, FEA), xp_map),
            pl.BlockSpec((blk, FEA), xc_map),
            pl.BlockSpec((halo, FEA), xn_map),
            pl.BlockSpec((9, FEA, HID), const3),
            pl.BlockSpec((1, HID), const2),
            pl.BlockSpec((1, HID), const2),
            pl.BlockSpec((HID, D), const2),
            pl.BlockSpec((1, D), const2),
            pl.BlockSpec((HID, D), const2),
            pl.BlockSpec((1, D), const2),
            pl.BlockSpec((NC, D), const2),
            pl.BlockSpec((NC, 1), const2),
            pl.BlockSpec((NC, HID), const2),
            pl.BlockSpec((NC, 1), const2),
        ],
        out_specs=[
            pl.BlockSpec((N, NC), const2),
            pl.BlockSpec((1, NC), const2),
            pl.BlockSpec((1, 1), const2),
        ],
        out_shape=[
            jax.ShapeDtypeStruct((N, NC), jnp.float32),
            jax.ShapeDtypeStruct((1, NC), jnp.float32),
            jax.ShapeDtypeStruct((1, 1), jnp.int32),
        ],
        scratch_shapes=[
            pltpu.VMEM((N, HID), jnp.float32),
            pltpu.VMEM((2, HID), jnp.float32),
            pltpu.VMEM((4, N), jnp.float32),
        ],
    )(h, h, h, w_taps, bn_gamma.reshape(1, HID), bn_beta.reshape(1, HID),
      Wa.T, ba.reshape(1, D), Wb.T, bb.reshape(1, D),
      Wc, bc.reshape(NC, 1), Wcls, bcls.reshape(NC, 1))
    return fs, yp.reshape(NC), yh.reshape(1)


# speedup vs baseline: 1.0925x; 1.0925x over previous
import functools

import jax
import jax.numpy as jnp
from jax.experimental import pallas as pl
from jax.experimental.pallas import tpu as pltpu


def _fused_kernel(xp_ref, xc_ref, xn_ref, w_ref, g_ref, b_ref,
                  wa_ref, ba_ref, wb_ref, bb_ref, wc_ref, bc_ref,
                  wcls_ref, bcls_ref,
                  fs_ref, yp_ref, yh_ref,
                  y_sc, st_sc, dc_sc, *, S, blk, nblk):
    p = pl.program_id(0)
    i = pl.program_id(1)

    @pl.when(p == 0)
    def _conv_phase():
        xp = xp_ref[...] * (i > 0).astype(jnp.float32)
        xn = xn_ref[...] * (i < nblk - 1).astype(jnp.float32)
        qs = []
        for dc in range(3):
            a = jnp.dot(xc_ref[...], w_ref[3 + dc],
                        preferred_element_type=jnp.float32)
            b0 = jnp.dot(xp, w_ref[dc], preferred_element_type=jnp.float32)
            b1 = jnp.dot(xc_ref[0:blk - 128, :], w_ref[dc],
                         preferred_element_type=jnp.float32)
            c0 = jnp.dot(xc_ref[128:blk, :], w_ref[6 + dc],
                         preferred_element_type=jnp.float32)
            c1 = jnp.dot(xn, w_ref[6 + dc],
                         preferred_element_type=jnp.float32)
            qs.append(a + jnp.concatenate([b0, b1], axis=0)
                      + jnp.concatenate([c0, c1], axis=0))
        c = jax.lax.broadcasted_iota(jnp.int32, (blk, 1), 0) % S
        y = (qs[1]
             + jnp.where(c != 0, jnp.roll(qs[0], 1, axis=0), 0.0)
             + jnp.where(c != S - 1, jnp.roll(qs[2], -1, axis=0), 0.0))
        y_sc[pl.ds(i * blk, blk), :] = y
        part = jnp.concatenate([jnp.sum(y, axis=0, keepdims=True),
                                jnp.sum(y * y, axis=0, keepdims=True)], axis=0)

        @pl.when(i == 0)
        def _init():
            st_sc[...] = part

        @pl.when(i > 0)
        def _acc():
            st_sc[...] = st_sc[...] + part

    @pl.when(p == 1)
    def _head_phase():
        n = nblk * blk
        mean = st_sc[0:1, :] / n
        var = st_sc[1:2, :] / n - mean * mean
        scale = jax.lax.rsqrt(var + 1e-5) * g_ref[...]
        y = y_sc[pl.ds(i * blk, blk), :]
        hh = jnp.maximum((y - mean) * scale + b_ref[...], 0.0)
        a = jnp.tanh(
            jnp.dot(hh, wa_ref[...], preferred_element_type=jnp.float32)
            + ba_ref[...])
        gate = jax.nn.sigmoid(
            jnp.dot(hh, wb_ref[...], preferred_element_type=jnp.float32)
            + bb_ref[...])
        ab = a * gate
        det = jax.lax.dot_general(
            wc_ref[...], ab, (((1,), (1,)), ((), ())),
            preferred_element_type=jnp.float32) + bc_ref[...]
        cls = jax.lax.dot_general(
            wcls_ref[...], hh, (((1,), (1,)), ((), ())),
            preferred_element_type=jnp.float32) + bcls_ref[...]
        cm = jnp.max(cls, axis=0, keepdims=True)
        ce = jnp.exp(cls - cm)
        cls_score = ce / jnp.sum(ce, axis=0, keepdims=True)
        dc_sc[0:2, pl.ds(i * blk, blk)] = det
        dc_sc[2:4, pl.ds(i * blk, blk)] = cls_score

        @pl.when(i == nblk - 1)
        def _finale():
            det_f = dc_sc[0:2, :]
            cls_f = dc_sc[2:4, :]
            dm = jnp.max(det_f, axis=1, keepdims=True)
            de = jnp.exp(det_f - dm)
            det_s = de / jnp.sum(de, axis=1, keepdims=True)
            fs = cls_f * det_s
            fs_ref[...] = fs
            yp = jnp.clip(jnp.sum(fs, axis=1, keepdims=True),
                          1e-10, 1.0 - 1e-10)
            yp_ref[...] = yp
            yh_ref[...] = (yp[1:2, :] > yp[0:1, :]).astype(jnp.int32)


def kernel(h, conv_w, bn_gamma, bn_beta, Wa, ba, Wb, bb, Wc, bc, Wcls, bcls):
    N, FEA = h.shape
    HID = conv_w.shape[0]
    D = Wa.shape[0]
    NC = Wc.shape[0]
    S = 128
    rows = 16
    blk = rows * S
    nblk = N // blk
    halo = 128
    nhalo = N // halo
    hpb = blk // halo
    w_taps = jnp.transpose(conv_w, (2, 3, 1, 0)).reshape(9, FEA, HID)

    def xp_map(p, i):
        return (jnp.where(p == 0, jnp.maximum(hpb * i - 1, 0),
                          hpb * (nblk - 1) - 1), 0)

    def xc_map(p, i):
        return (jnp.where(p == 0, i, nblk - 1), 0)

    def xn_map(p, i):
        return (jnp.where(p == 0, jnp.minimum(hpb * (i + 1), nhalo - 1),
                          nhalo - 1), 0)

    const2 = lambda p, i: (0, 0)
    const3 = lambda p, i: (0, 0, 0)

    fs, yp, yh = pl.pallas_call(
        functools.partial(_fused_kernel, S=S, blk=blk, nblk=nblk),
        grid=(2, nblk),
        in_specs=[
            pl.BlockSpec((halo, FEA), xp_map),
            pl.BlockSpec((blk, FEA), xc_map),
            pl.BlockSpec((halo, FEA), xn_map),
            pl.BlockSpec((9, FEA, HID), const3),
            pl.BlockSpec((1, HID), const2),
            pl.BlockSpec((1, HID), const2),
            pl.BlockSpec((HID, D), const2),
            pl.BlockSpec((1, D), const2),
            pl.BlockSpec((HID, D), const2),
            pl.BlockSpec((1, D), const2),
            pl.BlockSpec((NC, D), const2),
            pl.BlockSpec((NC, 1), const2),
            pl.BlockSpec((NC, HID), const2),
            pl.BlockSpec((NC, 1), const2),
        ],
        out_specs=[
            pl.BlockSpec((NC, N), const2),
            pl.BlockSpec((NC, 1), const2),
            pl.BlockSpec((1, 1), const2),
        ],
        out_shape=[
            jax.ShapeDtypeStruct((NC, N), jnp.float32),
            jax.ShapeDtypeStruct((NC, 1), jnp.float32),
            jax.ShapeDtypeStruct((1, 1), jnp.int32),
        ],
        scratch_shapes=[
            pltpu.VMEM((N, HID), jnp.float32),
            pltpu.VMEM((2, HID), jnp.float32),
            pltpu.VMEM((4, N), jnp.float32),
        ],
    )(h, h, h, w_taps, bn_gamma.reshape(1, HID), bn_beta.reshape(1, HID),
      Wa.T, ba.reshape(1, D), Wb.T, bb.reshape(1, D),
      Wc, bc.reshape(NC, 1), Wcls, bcls.reshape(NC, 1))
    return fs.T, yp.reshape(NC), yh.reshape(1)


# speedup vs baseline: 1.1175x; 1.0229x over previous
import functools

import jax
import jax.numpy as jnp
from jax.experimental import pallas as pl
from jax.experimental.pallas import tpu as pltpu


def _fused_kernel(xp_ref, xc_ref, xn_ref, w_ref, g_ref, b_ref,
                  wa_ref, ba_ref, wb_ref, bb_ref, wc_ref, bc_ref,
                  wcls_ref, bcls_ref,
                  fs_ref, yp_ref, yh_ref,
                  y_sc, st_sc, dc_sc, *, S, blk, nblk):
    p = pl.program_id(0)
    i = pl.program_id(1)

    @pl.when(p == 0)
    def _conv_phase():
        sub = blk // 2
        xp = xp_ref[120:256, :] * (i > 0).astype(jnp.float32)
        xn = xn_ref[0:136, :] * (i < nblk - 1).astype(jnp.float32)
        c = jax.lax.broadcasted_iota(jnp.int32, (sub, 1), 0) % S
        parts = []
        for s in range(2):
            if s == 0:
                cat = jnp.concatenate([xp, xc_ref[0:sub + 136, :]], axis=0)
            else:
                cat = jnp.concatenate([xc_ref[sub - 136:blk, :], xn], axis=0)
            cat = cat.astype(jnp.bfloat16)
            qs = []
            for dc in range(3):
                q = None
                for dr in range(3):
                    xs = cat[128 * dr:128 * dr + sub + 16, :]
                    d = jnp.dot(xs, w_ref[dr * 3 + dc],
                                preferred_element_type=jnp.float32)
                    q = d if q is None else q + d
                qs.append(q)
            ys = (qs[1][8:8 + sub, :]
                  + jnp.where(c != 0, qs[0][7:7 + sub, :], 0.0)
                  + jnp.where(c != S - 1, qs[2][9:9 + sub, :], 0.0))
            y_sc[pl.ds(i * blk + s * sub, sub), :] = ys
            parts.append(
                jnp.concatenate([jnp.sum(ys, axis=0, keepdims=True),
                                 jnp.sum(ys * ys, axis=0, keepdims=True)],
                                axis=0))
        part = parts[0] + parts[1]

        @pl.when(i == 0)
        def _init():
            st_sc[...] = part

        @pl.when(i > 0)
        def _acc():
            st_sc[...] = st_sc[...] + part

    @pl.when(p == 1)
    def _head_phase():
        n = nblk * blk
        mean = st_sc[0:1, :] / n
        var = st_sc[1:2, :] / n - mean * mean
        scale = jax.lax.rsqrt(var + 1e-5) * g_ref[...]
        y = y_sc[pl.ds(i * blk, blk), :]
        hh = jnp.maximum((y - mean) * scale + b_ref[...], 0.0)
        a = jnp.tanh(
            jnp.dot(hh, wa_ref[...], preferred_element_type=jnp.float32)
            + ba_ref[...])
        gate = jax.nn.sigmoid(
            jnp.dot(hh, wb_ref[...], preferred_element_type=jnp.float32)
            + bb_ref[...])
        ab = a * gate
        det = jax.lax.dot_general(
            wc_ref[...], ab, (((1,), (1,)), ((), ())),
            preferred_element_type=jnp.float32) + bc_ref[...]
        cls = jax.lax.dot_general(
            wcls_ref[...], hh, (((1,), (1,)), ((), ())),
            preferred_element_type=jnp.float32) + bcls_ref[...]
        cm = jnp.max(cls, axis=0, keepdims=True)
        ce = jnp.exp(cls - cm)
        cls_score = ce / jnp.sum(ce, axis=0, keepdims=True)
        dc_sc[0:2, pl.ds(i * blk, blk)] = det
        dc_sc[2:4, pl.ds(i * blk, blk)] = cls_score

        @pl.when(i == nblk - 1)
        def _finale():
            det_f = dc_sc[0:2, :]
            cls_f = dc_sc[2:4, :]
            dm = jnp.max(det_f, axis=1, keepdims=True)
            de = jnp.exp(det_f - dm)
            det_s = de / jnp.sum(de, axis=1, keepdims=True)
            fs = cls_f * det_s
            fs_ref[...] = fs
            yp = jnp.clip(jnp.sum(fs, axis=1, keepdims=True),
                          1e-10, 1.0 - 1e-10)
            yp_ref[...] = yp
            yh_ref[...] = (yp[1:2, :] > yp[0:1, :]).astype(jnp.int32)


def kernel(h, conv_w, bn_gamma, bn_beta, Wa, ba, Wb, bb, Wc, bc, Wcls, bcls):
    N, FEA = h.shape
    HID = conv_w.shape[0]
    D = Wa.shape[0]
    NC = Wc.shape[0]
    S = 128
    rows = 16
    blk = rows * S
    nblk = N // blk
    halo = 256
    nhalo = N // halo
    hpb = blk // halo
    w_taps = jnp.transpose(conv_w, (2, 3, 1, 0)).reshape(9, FEA, HID)
    w_taps = w_taps.astype(jnp.bfloat16)

    def xp_map(p, i):
        return (jnp.where(p == 0, jnp.maximum(hpb * i - 1, 0),
                          hpb * (nblk - 1) - 1), 0)

    def xc_map(p, i):
        return (jnp.where(p == 0, i, nblk - 1), 0)

    def xn_map(p, i):
        return (jnp.where(p == 0, jnp.minimum(hpb * (i + 1), nhalo - 1),
                          nhalo - 1), 0)

    const2 = lambda p, i: (0, 0)
    const3 = lambda p, i: (0, 0, 0)

    fs, yp, yh = pl.pallas_call(
        functools.partial(_fused_kernel, S=S, blk=blk, nblk=nblk),
        grid=(2, nblk),
        in_specs=[
            pl.BlockSpec((halo, FEA), xp_map),
            pl.BlockSpec((blk, FEA), xc_map),
            pl.BlockSpec((halo, FEA), xn_map),
            pl.BlockSpec((9, FEA, HID), const3),
            pl.BlockSpec((1, HID), const2),
            pl.BlockSpec((1, HID), const2),
            pl.BlockSpec((HID, D), const2),
            pl.BlockSpec((1, D), const2),
            pl.BlockSpec((HID, D), const2),
            pl.BlockSpec((1, D), const2),
            pl.BlockSpec((NC, D), const2),
            pl.BlockSpec((NC, 1), const2),
            pl.BlockSpec((NC, HID), const2),
            pl.BlockSpec((NC, 1), const2),
        ],
        out_specs=[
            pl.BlockSpec((NC, N), const2),
            pl.BlockSpec((NC, 1), const2),
            pl.BlockSpec((1, 1), const2),
        ],
        out_shape=[
            jax.ShapeDtypeStruct((NC, N), jnp.float32),
            jax.ShapeDtypeStruct((NC, 1), jnp.float32),
            jax.ShapeDtypeStruct((1, 1), jnp.int32),
        ],
        scratch_shapes=[
            pltpu.VMEM((N, HID), jnp.float32),
            pltpu.VMEM((2, HID), jnp.float32),
            pltpu.VMEM((4, N), jnp.float32),
        ],
    )(h, h, h, w_taps, bn_gamma.reshape(1, HID), bn_beta.reshape(1, HID),
      Wa.T, ba.reshape(1, D), Wb.T, bb.reshape(1, D),
      Wc, bc.reshape(NC, 1), Wcls, bcls.reshape(NC, 1))
    return fs.T, yp.reshape(NC), yh.reshape(1)


# speedup vs baseline: 1.1233x; 1.0052x over previous
import functools

import jax
import jax.numpy as jnp
from jax.experimental import pallas as pl
from jax.experimental.pallas import tpu as pltpu


def _fused_kernel(xp_ref, xc_ref, xn_ref, w_ref, g_ref, b_ref,
                  wa_ref, ba_ref, wb_ref, bb_ref, wc_ref, bc_ref,
                  wcls_ref, bcls_ref,
                  fs_ref, yp_ref, yh_ref,
                  y_sc, st_sc, dc_sc, *, S, blk, nblk):
    p = pl.program_id(0)
    i = pl.program_id(1)

    @pl.when(p == 0)
    def _conv_phase():
        xp = xp_ref[120:256, :] * (i > 0).astype(jnp.float32)
        xn = xn_ref[0:136, :] * (i < nblk - 1).astype(jnp.float32)
        x_cat = jnp.concatenate([xp, xc_ref[...], xn],
                                axis=0).astype(jnp.bfloat16)
        qs = []
        for dc in range(3):
            q = None
            for dr in range(3):
                xs = x_cat[128 * dr:128 * dr + blk + 16, :]
                d = jnp.dot(xs, w_ref[dr * 3 + dc],
                            preferred_element_type=jnp.float32)
                q = d if q is None else q + d
            qs.append(q)
        c = jax.lax.broadcasted_iota(jnp.int32, (blk, 1), 0) % S
        y = (qs[1][8:8 + blk, :]
             + jnp.where(c != 0, qs[0][7:7 + blk, :], 0.0)
             + jnp.where(c != S - 1, qs[2][9:9 + blk, :], 0.0))
        y_sc[pl.ds(i * blk, blk), :] = y
        part = jnp.concatenate([jnp.sum(y, axis=0, keepdims=True),
                                jnp.sum(y * y, axis=0, keepdims=True)], axis=0)

        @pl.when(i == 0)
        def _init():
            st_sc[...] = part

        @pl.when(i > 0)
        def _acc():
            st_sc[...] = st_sc[...] + part

    @pl.when(p == 1)
    def _head_phase():
        n = nblk * blk
        mean = st_sc[0:1, :] / n
        var = st_sc[1:2, :] / n - mean * mean
        scale = jax.lax.rsqrt(var + 1e-5) * g_ref[...]
        y = y_sc[pl.ds(i * blk, blk), :]
        hh = jnp.maximum((y - mean) * scale + b_ref[...], 0.0)
        a = jnp.tanh(
            jnp.dot(hh, wa_ref[...], preferred_element_type=jnp.float32)
            + ba_ref[...])
        gate = jax.nn.sigmoid(
            jnp.dot(hh, wb_ref[...], preferred_element_type=jnp.float32)
            + bb_ref[...])
        ab = a * gate
        det = jax.lax.dot_general(
            wc_ref[...], ab, (((1,), (1,)), ((), ())),
            preferred_element_type=jnp.float32) + bc_ref[...]
        cls = jax.lax.dot_general(
            wcls_ref[...], hh, (((1,), (1,)), ((), ())),
            preferred_element_type=jnp.float32) + bcls_ref[...]
        cm = jnp.max(cls, axis=0, keepdims=True)
        ce = jnp.exp(cls - cm)
        cls_score = ce / jnp.sum(ce, axis=0, keepdims=True)
        dc_sc[0:2, pl.ds(i * blk, blk)] = det
        dc_sc[2:4, pl.ds(i * blk, blk)] = cls_score

        @pl.when(i == nblk - 1)
        def _finale():
            det_f = dc_sc[0:2, :]
            cls_f = dc_sc[2:4, :]
            dm = jnp.max(det_f, axis=1, keepdims=True)
            de = jnp.exp(det_f - dm)
            det_s = de / jnp.sum(de, axis=1, keepdims=True)
            fs = cls_f * det_s
            fs_ref[...] = fs
            yp = jnp.clip(jnp.sum(fs, axis=1, keepdims=True),
                          1e-10, 1.0 - 1e-10)
            yp_ref[...] = yp
            yh_ref[...] = (yp[1:2, :] > yp[0:1, :]).astype(jnp.int32)


def kernel(h, conv_w, bn_gamma, bn_beta, Wa, ba, Wb, bb, Wc, bc, Wcls, bcls):
    N, FEA = h.shape
    HID = conv_w.shape[0]
    D = Wa.shape[0]
    NC = Wc.shape[0]
    S = 128
    rows = 16
    blk = rows * S
    nblk = N // blk
    halo = 256
    nhalo = N // halo
    hpb = blk // halo
    w_taps = jnp.transpose(conv_w, (2, 3, 1, 0)).reshape(9, FEA, HID)
    w_taps = w_taps.astype(jnp.bfloat16)

    def xp_map(p, i):
        return (jnp.where(p == 0, jnp.maximum(hpb * i - 1, 0),
                          hpb * (nblk - 1) - 1), 0)

    def xc_map(p, i):
        return (jnp.where(p == 0, i, nblk - 1), 0)

    def xn_map(p, i):
        return (jnp.where(p == 0, jnp.minimum(hpb * (i + 1), nhalo - 1),
                          nhalo - 1), 0)

    const2 = lambda p, i: (0, 0)
    const3 = lambda p, i: (0, 0, 0)

    fs, yp, yh = pl.pallas_call(
        functools.partial(_fused_kernel, S=S, blk=blk, nblk=nblk),
        grid=(2, nblk),
        in_specs=[
            pl.BlockSpec((halo, FEA), xp_map),
            pl.BlockSpec((blk, FEA), xc_map),
            pl.BlockSpec((halo, FEA), xn_map),
            pl.BlockSpec((9, FEA, HID), const3),
            pl.BlockSpec((1, HID), const2),
            pl.BlockSpec((1, HID), const2),
            pl.BlockSpec((HID, D), const2),
            pl.BlockSpec((1, D), const2),
            pl.BlockSpec((HID, D), const2),
            pl.BlockSpec((1, D), const2),
            pl.BlockSpec((NC, D), const2),
            pl.BlockSpec((NC, 1), const2),
            pl.BlockSpec((NC, HID), const2),
            pl.BlockSpec((NC, 1), const2),
        ],
        out_specs=[
            pl.BlockSpec((NC, N), const2),
            pl.BlockSpec((NC, 1), const2),
            pl.BlockSpec((1, 1), const2),
        ],
        out_shape=[
            jax.ShapeDtypeStruct((NC, N), jnp.float32),
            jax.ShapeDtypeStruct((NC, 1), jnp.float32),
            jax.ShapeDtypeStruct((1, 1), jnp.int32),
        ],
        scratch_shapes=[
            pltpu.VMEM((N, HID), jnp.float32),
            pltpu.VMEM((2, HID), jnp.float32),
            pltpu.VMEM((4, N), jnp.float32),
        ],
    )(h, h, h, w_taps, bn_gamma.reshape(1, HID), bn_beta.reshape(1, HID),
      Wa.T, ba.reshape(1, D), Wb.T, bb.reshape(1, D),
      Wc, bc.reshape(NC, 1), Wcls, bcls.reshape(NC, 1))
    return fs.T, yp.reshape(NC), yh.reshape(1)
